# baseline (speedup 1.0000x reference)
import functools

import jax
import jax.numpy as jnp
from jax import lax
from jax.experimental import pallas as pl
from jax.experimental.pallas import tpu as pltpu
from jax.experimental.pallas import tpu_sc as plsc

VOCAB = 1000000
EMBED = 64
BATCH = 16384

NC = 2
NS = 16
L = 16
NW = NC * NS
BPW = BATCH // NW
GP = 16
NGRP = BPW // GP
ROWS_G = GP * EMBED
NCHUNK = ROWS_G // 128
GRAN = VOCAB // L


def _sc_body(center_hbm, context_hbm, At, Bt, score_hbm,
             cidx_v, xidx_v, aidx, bidx, adst, bdst, score_v, sem):
  wid = lax.axis_index("s") * NC + lax.axis_index("c")

  pltpu.sync_copy(center_hbm.at[wid], cidx_v)
  pltpu.sync_copy(context_hbm.at[wid], xidx_v)

  lanes = lax.iota(jnp.int32, L)

  def group_body(g, _):
    cvec = cidx_v[g, :]
    xvec = xidx_v[g, :]
    cbase = lax.shift_right_logical(cvec, 4)
    xbase = lax.shift_right_logical(xvec, 4)
    for j in range(EMBED):
      aidx[pl.ds(j * GP, GP)] = cbase + (j * GRAN)
      bidx[pl.ds(j * GP, GP)] = xbase + (j * GRAN)
    copies = []
    for c in range(NCHUNK):
      sl = pl.ds(c * 128, 128)
      copies.append(pltpu.async_copy(At.at[aidx.at[sl]], adst.at[sl], sem))
      copies.append(pltpu.async_copy(Bt.at[bidx.at[sl]], bdst.at[sl], sem))
    for cp in copies:
      cp.wait()
    clow = jnp.bitwise_and(cvec, L - 1)
    xlow = jnp.bitwise_and(xvec, L - 1)
    acc = None
    for j in range(EMBED):
      rowv = lanes + (j * GP)
      av = plsc.load_gather(adst, [rowv, clow])
      bv = plsc.load_gather(bdst, [rowv, xlow])
      acc = av * bv if acc is None else acc + av * bv
    score_v[g, :] = acc
    return _

  lax.fori_loop(0, NGRP, group_body, None)

  pltpu.sync_copy(score_v, score_hbm.at[wid])


_sc_partials = pl.kernel(
    _sc_body,
    out_type=jax.ShapeDtypeStruct((NW, NGRP, GP), jnp.float32),
    mesh=plsc.VectorSubcoreMesh(core_axis_name="c", subcore_axis_name="s",
                                num_cores=NC, num_subcores=NS),
    scratch_types=[
        pltpu.VMEM((NGRP, GP), jnp.int32),
        pltpu.VMEM((NGRP, GP), jnp.int32),
        pltpu.VMEM((ROWS_G,), jnp.int32),
        pltpu.VMEM((ROWS_G,), jnp.int32),
        pltpu.VMEM((ROWS_G, L), jnp.float32),
        pltpu.VMEM((ROWS_G, L), jnp.float32),
        pltpu.VMEM((NGRP, GP), jnp.float32),
        pltpu.SemaphoreType.DMA,
    ],
    compiler_params=pltpu.CompilerParams(use_tc_tiling_on_sc=False,
                                         needs_layout_passes=False),
)


def _tc_body(score_ref, o_ref):
  s = score_ref[...]
  lp = jnp.minimum(s, 0.0) - jnp.log1p(jnp.exp(-jnp.abs(s)))
  o_ref[0, 0] = -jnp.sum(lp) / BATCH


def kernel(center, context, input_embed, output_embed):
  At = input_embed.T.reshape(VOCAB * EMBED // L, L)
  Bt = output_embed.T.reshape(VOCAB * EMBED // L, L)
  c3 = center.astype(jnp.int32).reshape(NW, NGRP, GP)
  x3 = context.astype(jnp.int32).reshape(NW, NGRP, GP)
  score = _sc_partials(c3, x3, At, Bt)
  out = pl.pallas_call(
      _tc_body,
      out_shape=jax.ShapeDtypeStruct((1, 1), jnp.float32),
      out_specs=pl.BlockSpec(memory_space=pltpu.SMEM),
  )(score.reshape(BATCH // 128, 128))
  return out[0, 0]


# speedup vs baseline: 8.7559x; 8.7559x over previous
import functools

import jax
import jax.numpy as jnp
from jax import lax
from jax.experimental import pallas as pl
from jax.experimental.pallas import tpu as pltpu
from jax.experimental.pallas import tpu_sc as plsc

VOCAB = 1000000
EMBED = 64
BATCH = 16384

NC = 2
NS = 16
L = 16
NW = NC * NS
BPW = BATCH // NW
GP = 16
NGRP = BPW // GP
ROWS_G = GP * EMBED
NCHUNK = ROWS_G // 128
GRAN = VOCAB // L


def _sc_body(center_hbm, context_hbm, At, Bt, score_hbm,
             cidx_v, xidx_v, aidx, bidx, adst, bdst, score_v, sem):
  wid = lax.axis_index("s") * NC + lax.axis_index("c")

  pltpu.sync_copy(center_hbm.at[wid], cidx_v)
  pltpu.sync_copy(context_hbm.at[wid], xidx_v)

  lanes = lax.iota(jnp.int32, L)

  def group_body(g, _):
    cvec = cidx_v[g, :]
    xvec = xidx_v[g, :]
    aidx[...] = lax.shift_right_logical(cvec, 1)
    bidx[...] = lax.shift_right_logical(xvec, 1)
    cp1 = pltpu.async_copy(At.at[aidx], adst, sem)
    cp2 = pltpu.async_copy(Bt.at[bidx], bdst, sem)
    cp1.wait()
    cp2.wait()
    choff = jnp.bitwise_and(cvec, 1) * EMBED
    xhoff = jnp.bitwise_and(xvec, 1) * EMBED
    acc = None
    for j in range(EMBED):
      av = plsc.load_gather(adst, [lanes, choff + j])
      bv = plsc.load_gather(bdst, [lanes, xhoff + j])
      acc = av * bv if acc is None else acc + av * bv
    score_v[g, :] = acc
    return _

  lax.fori_loop(0, NGRP, group_body, None)

  pltpu.sync_copy(score_v, score_hbm.at[wid])


_sc_partials = pl.kernel(
    _sc_body,
    out_type=jax.ShapeDtypeStruct((NW, NGRP, GP), jnp.float32),
    mesh=plsc.VectorSubcoreMesh(core_axis_name="c", subcore_axis_name="s",
                                num_cores=NC, num_subcores=NS),
    scratch_types=[
        pltpu.VMEM((NGRP, GP), jnp.int32),
        pltpu.VMEM((NGRP, GP), jnp.int32),
        pltpu.VMEM((GP,), jnp.int32),
        pltpu.VMEM((GP,), jnp.int32),
        pltpu.VMEM((GP, 2 * EMBED), jnp.float32),
        pltpu.VMEM((GP, 2 * EMBED), jnp.float32),
        pltpu.VMEM((NGRP, GP), jnp.float32),
        pltpu.SemaphoreType.DMA,
    ],
    compiler_params=pltpu.CompilerParams(use_tc_tiling_on_sc=True,
                                         needs_layout_passes=False),
)


def _tc_body(score_ref, o_ref):
  s = score_ref[...]
  lp = jnp.minimum(s, 0.0) - jnp.log1p(jnp.exp(-jnp.abs(s)))
  o_ref[0, 0] = -jnp.sum(lp) / BATCH


def kernel(center, context, input_embed, output_embed):
  At = input_embed.reshape(VOCAB // 2, 2 * EMBED)
  Bt = output_embed.reshape(VOCAB // 2, 2 * EMBED)
  c3 = center.astype(jnp.int32).reshape(NW, NGRP, GP)
  x3 = context.astype(jnp.int32).reshape(NW, NGRP, GP)
  score = _sc_partials(c3, x3, At, Bt)
  out = pl.pallas_call(
      _tc_body,
      out_shape=jax.ShapeDtypeStruct((1, 1), jnp.float32),
      out_specs=pl.BlockSpec(memory_space=pltpu.SMEM),
  )(score.reshape(BATCH // 128, 128))
  return out[0, 0]


# speedup vs baseline: 17.6102x; 2.0112x over previous
import functools

import jax
import jax.numpy as jnp
from jax import lax
from jax.experimental import pallas as pl
from jax.experimental.pallas import tpu as pltpu
from jax.experimental.pallas import tpu_sc as plsc

VOCAB = 1000000
EMBED = 64
BATCH = 16384

NC = 2
NS = 16
L = 16
NW = NC * NS
BPW = BATCH // NW
NGRP = BPW // L
NCHUNK = BPW // 128


def _stripe_body(sa_hbm, sb_hbm, at_hbm, bt_hbm, rowsa_hbm, rowsb_hbm,
                 sidx, stripe, rows_v, sem):
  wid = lax.axis_index("s") * NC + lax.axis_index("c")
  lanes = lax.iota(jnp.int32, L)

  def one_table(idx_hbm, xt_hbm, rows_hbm):
    pltpu.sync_copy(idx_hbm.at[wid], sidx)

    def group_body(g, last_tc):
      ivec = sidx[g, :]
      tcvec = lax.shift_right_logical(ivec, 7)
      colvec = jnp.bitwise_and(ivec, 127)
      for k in range(L):
        msk = lanes == k
        tc = jnp.max(jnp.where(msk, tcvec, 0))
        col = jnp.max(jnp.where(msk, colvec, 0))

        @pl.when(tc != last_tc)
        def _fetch():
          off = pl.multiple_of(tc * 128, 128)
          pltpu.async_copy(xt_hbm.at[:, pl.ds(off, 128)], stripe, sem).wait()

        csplat = jnp.broadcast_to(col, (L,))
        for q in range(EMBED // L):
          v = plsc.load_gather(stripe, [lanes + q * L, csplat])
          rows_v[g * L + k, pl.ds(q * L, L)] = v
        last_tc = tc
      return last_tc

    lax.fori_loop(0, NGRP, group_body, jnp.int32(-1))
    pltpu.sync_copy(rows_v, rows_hbm.at[pl.ds(wid * BPW, BPW)])

  one_table(sa_hbm, at_hbm, rowsa_hbm)
  one_table(sb_hbm, bt_hbm, rowsb_hbm)


_stripe_gather = pl.kernel(
    _stripe_body,
    out_type=(jax.ShapeDtypeStruct((BATCH, EMBED), jnp.float32),
              jax.ShapeDtypeStruct((BATCH, EMBED), jnp.float32)),
    mesh=plsc.VectorSubcoreMesh(core_axis_name="c", subcore_axis_name="s",
                                num_cores=NC, num_subcores=NS),
    scratch_types=[
        pltpu.VMEM((NGRP, L), jnp.int32),
        pltpu.VMEM((EMBED, 128), jnp.float32),
        pltpu.VMEM((BPW, EMBED), jnp.float32),
        pltpu.SemaphoreType.DMA,
    ],
    compiler_params=pltpu.CompilerParams(use_tc_tiling_on_sc=True,
                                         needs_layout_passes=False),
)


def _pair_body(ia_hbm, ib_hbm, rowsa_hbm, rowsb_hbm, part_hbm,
               ia_v, ib_v, arows, brows, part_v, sem):
  wid = lax.axis_index("s") * NC + lax.axis_index("c")
  pltpu.sync_copy(ia_hbm.at[wid], ia_v)
  pltpu.sync_copy(ib_hbm.at[wid], ib_v)
  copies = []
  for c in range(NCHUNK):
    sl = pl.ds(c * 128, 128)
    copies.append(pltpu.async_copy(rowsa_hbm.at[ia_v.at[c]], arows.at[sl], sem))
    copies.append(pltpu.async_copy(rowsb_hbm.at[ib_v.at[c]], brows.at[sl], sem))
  for cp in copies:
    cp.wait()

  def row_body(r, _):
    acc = arows[r, pl.ds(0, L)] * brows[r, pl.ds(0, L)]
    for q in range(1, EMBED // L):
      acc = acc + arows[r, pl.ds(q * L, L)] * brows[r, pl.ds(q * L, L)]
    part_v[r, :] = acc
    return _

  lax.fori_loop(0, BPW, row_body, None)
  pltpu.sync_copy(part_v, part_hbm.at[pl.ds(wid * BPW, BPW)])


_pair_dot = pl.kernel(
    _pair_body,
    out_type=jax.ShapeDtypeStruct((BATCH, L), jnp.float32),
    mesh=plsc.VectorSubcoreMesh(core_axis_name="c", subcore_axis_name="s",
                                num_cores=NC, num_subcores=NS),
    scratch_types=[
        pltpu.VMEM((NCHUNK, 128), jnp.int32),
        pltpu.VMEM((NCHUNK, 128), jnp.int32),
        pltpu.VMEM((BPW, EMBED), jnp.float32),
        pltpu.VMEM((BPW, EMBED), jnp.float32),
        pltpu.VMEM((BPW, L), jnp.float32),
        pltpu.SemaphoreType.DMA,
    ],
    compiler_params=pltpu.CompilerParams(use_tc_tiling_on_sc=False),
)


def _tc_body(part_ref, o_ref):
  s = jnp.sum(part_ref[...], axis=1)
  lp = jnp.minimum(s, 0.0) - jnp.log1p(jnp.exp(-jnp.abs(s)))
  o_ref[0, 0] = -jnp.sum(lp) / BATCH


def kernel(center, context, input_embed, output_embed):
  center = center.astype(jnp.int32)
  context = context.astype(jnp.int32)
  pa = jnp.argsort(center)
  pb = jnp.argsort(context)
  sa = jnp.take(center, pa)
  sb = jnp.take(context, pb)
  inv_pa = jnp.argsort(pa).astype(jnp.int32)
  inv_pb = jnp.argsort(pb).astype(jnp.int32)
  rows_a, rows_b = _stripe_gather(
      sa.reshape(NW, NGRP, L), sb.reshape(NW, NGRP, L),
      input_embed.T, output_embed.T)
  part = _pair_dot(inv_pa.reshape(NW, NCHUNK, 128),
                   inv_pb.reshape(NW, NCHUNK, 128), rows_a, rows_b)
  out = pl.pallas_call(
      _tc_body,
      out_shape=jax.ShapeDtypeStruct((1, 1), jnp.float32),
      out_specs=pl.BlockSpec(memory_space=pltpu.SMEM),
  )(part)
  return out[0, 0]


# speedup vs baseline: 18.6171x; 1.0572x over previous
import functools

import jax
import jax.numpy as jnp
from jax import lax
from jax.experimental import pallas as pl
from jax.experimental.pallas import tpu as pltpu
from jax.experimental.pallas import tpu_sc as plsc

VOCAB = 1000000
EMBED = 64
BATCH = 16384

NC = 2
NS = 16
L = 16
NW = NC * NS
BPW = BATCH // NW
NGRP = BPW // L
NCHUNK = BPW // 128


NRUN = BPW + L


def _stripe_body(sa_hbm, sb_hbm, rta_hbm, rsa_hbm, na_hbm,
                 rtb_hbm, rsb_hbm, nb_hbm, at_hbm, bt_hbm,
                 rowsa_hbm, rowsb_hbm,
                 sidx, rtc_v, rst_v, n_v, s0, s1, rows_v, sem0, sem1):
  wid = lax.axis_index("s") * NC + lax.axis_index("c")
  lanes = lax.iota(jnp.int32, L)

  def vscal(chunk, lane):
    return jnp.max(jnp.where(lanes == lane, chunk, 0))

  def one_table(idx_hbm, rt_hbm, rs_hbm, n_hbm, xt_hbm, rows_hbm):
    pltpu.sync_copy(idx_hbm.at[wid], sidx)
    pltpu.sync_copy(rt_hbm.at[wid], rtc_v)
    pltpu.sync_copy(rs_hbm.at[wid], rst_v)
    pltpu.sync_copy(n_hbm.at[wid], n_v)
    nrun = vscal(n_v[...], 0)

    def rscal(ref, r):
      base = pl.multiple_of(jnp.bitwise_and(r, -L), 8)
      return vscal(ref[pl.ds(base, L)], jnp.bitwise_and(r, L - 1))

    def fetch(r, slot_ref, sem):
      tc = rscal(rtc_v, r)
      off = pl.multiple_of(tc * 128, 128)
      pltpu.async_copy(xt_hbm.at[:, pl.ds(off, 128)], slot_ref, sem)

    def drain(slot_ref, sem):
      pltpu.make_async_copy(xt_hbm.at[:, pl.ds(0, 128)], slot_ref, sem).wait()

    def extract(r, slot_ref):
      p0 = rscal(rst_v, r)
      p1 = rscal(rst_v, r + 1)

      def pos_body(p, _):
        ivec = sidx[lax.shift_right_logical(p, 4), :]
        col = vscal(jnp.bitwise_and(ivec, 127), jnp.bitwise_and(p, L - 1))
        csplat = jnp.broadcast_to(col, (L,))
        for q in range(EMBED // L):
          v = plsc.load_gather(slot_ref, [lanes + q * L, csplat])
          rows_v[p, pl.ds(q * L, L)] = v
        return _

      lax.fori_loop(p0, p1, pos_body, None)

    fetch(0, s0, sem0)

    def run_pair(t, _):
      r0 = 2 * t
      r1 = r0 + 1

      @pl.when(r1 < nrun)
      def _pf1():
        fetch(r1, s1, sem1)

      drain(s0, sem0)
      extract(r0, s0)

      @pl.when(r1 + 1 < nrun)
      def _pf0():
        fetch(r1 + 1, s0, sem0)

      @pl.when(r1 < nrun)
      def _do1():
        drain(s1, sem1)
        extract(r1, s1)

      return _

    lax.fori_loop(0, lax.div(nrun + 1, 2), run_pair, None)
    pltpu.sync_copy(rows_v, rows_hbm.at[pl.ds(wid * BPW, BPW)])

  one_table(sa_hbm, rta_hbm, rsa_hbm, na_hbm, at_hbm, rowsa_hbm)
  one_table(sb_hbm, rtb_hbm, rsb_hbm, nb_hbm, bt_hbm, rowsb_hbm)


_stripe_gather = pl.kernel(
    _stripe_body,
    out_type=(jax.ShapeDtypeStruct((BATCH, EMBED), jnp.float32),
              jax.ShapeDtypeStruct((BATCH, EMBED), jnp.float32)),
    mesh=plsc.VectorSubcoreMesh(core_axis_name="c", subcore_axis_name="s",
                                num_cores=NC, num_subcores=NS),
    scratch_types=[
        pltpu.VMEM((NGRP, L), jnp.int32),
        pltpu.VMEM((NRUN,), jnp.int32),
        pltpu.VMEM((NRUN,), jnp.int32),
        pltpu.VMEM((L,), jnp.int32),
        pltpu.VMEM((EMBED, 128), jnp.float32),
        pltpu.VMEM((EMBED, 128), jnp.float32),
        pltpu.VMEM((BPW, EMBED), jnp.float32),
        pltpu.SemaphoreType.DMA,
        pltpu.SemaphoreType.DMA,
    ],
    compiler_params=pltpu.CompilerParams(use_tc_tiling_on_sc=True,
                                         needs_layout_passes=False),
)


def _run_tables(sorted_idx):
  t2 = lax.shift_right_logical(sorted_idx, 7).reshape(NW, BPW)
  new = jnp.concatenate(
      [jnp.ones((NW, 1), bool), t2[:, 1:] != t2[:, :-1]], axis=1)
  runid = jnp.cumsum(new, axis=1).astype(jnp.int32) - 1
  nrun = runid[:, -1] + 1
  w_ix = jnp.broadcast_to(jnp.arange(NW)[:, None], (NW, BPW))
  p_ix = jnp.broadcast_to(jnp.arange(BPW, dtype=jnp.int32)[None], (NW, BPW))
  run_tc = jnp.zeros((NW, NRUN), jnp.int32).at[w_ix, runid].set(t2)
  run_st = jnp.full((NW, NRUN), BPW, jnp.int32).at[
      w_ix, runid].min(p_ix, mode="drop")
  nrep = jnp.broadcast_to(nrun[:, None], (NW, L)).astype(jnp.int32)
  return run_tc, run_st, nrep


def _pair_body(ia_hbm, ib_hbm, rowsa_hbm, rowsb_hbm, part_hbm,
               ia_v, ib_v, arows, brows, part_v, sem):
  wid = lax.axis_index("s") * NC + lax.axis_index("c")
  pltpu.sync_copy(ia_hbm.at[wid], ia_v)
  pltpu.sync_copy(ib_hbm.at[wid], ib_v)
  copies = []
  for c in range(NCHUNK):
    sl = pl.ds(c * 128, 128)
    copies.append(pltpu.async_copy(rowsa_hbm.at[ia_v.at[c]], arows.at[sl], sem))
    copies.append(pltpu.async_copy(rowsb_hbm.at[ib_v.at[c]], brows.at[sl], sem))
  for cp in copies:
    cp.wait()

  def row_body(r, _):
    acc = arows[r, pl.ds(0, L)] * brows[r, pl.ds(0, L)]
    for q in range(1, EMBED // L):
      acc = acc + arows[r, pl.ds(q * L, L)] * brows[r, pl.ds(q * L, L)]
    part_v[r, :] = acc
    return _

  lax.fori_loop(0, BPW, row_body, None)
  pltpu.sync_copy(part_v, part_hbm.at[pl.ds(wid * BPW, BPW)])


_pair_dot = pl.kernel(
    _pair_body,
    out_type=jax.ShapeDtypeStruct((BATCH, L), jnp.float32),
    mesh=plsc.VectorSubcoreMesh(core_axis_name="c", subcore_axis_name="s",
                                num_cores=NC, num_subcores=NS),
    scratch_types=[
        pltpu.VMEM((NCHUNK, 128), jnp.int32),
        pltpu.VMEM((NCHUNK, 128), jnp.int32),
        pltpu.VMEM((BPW, EMBED), jnp.float32),
        pltpu.VMEM((BPW, EMBED), jnp.float32),
        pltpu.VMEM((BPW, L), jnp.float32),
        pltpu.SemaphoreType.DMA,
    ],
    compiler_params=pltpu.CompilerParams(use_tc_tiling_on_sc=False),
)


def _tc_body(part_ref, o_ref):
  s = jnp.sum(part_ref[...], axis=1)
  lp = jnp.minimum(s, 0.0) - jnp.log1p(jnp.exp(-jnp.abs(s)))
  o_ref[0, 0] = -jnp.sum(lp) / BATCH


def kernel(center, context, input_embed, output_embed):
  center = center.astype(jnp.int32)
  context = context.astype(jnp.int32)
  pa = jnp.argsort(center)
  pb = jnp.argsort(context)
  sa = jnp.take(center, pa)
  sb = jnp.take(context, pb)
  inv_pa = jnp.argsort(pa).astype(jnp.int32)
  inv_pb = jnp.argsort(pb).astype(jnp.int32)
  rta, rsa, na = _run_tables(sa)
  rtb, rsb, nb = _run_tables(sb)
  rows_a, rows_b = _stripe_gather(
      sa.reshape(NW, NGRP, L), sb.reshape(NW, NGRP, L),
      rta, rsa, na, rtb, rsb, nb,
      input_embed.T, output_embed.T)
  part = _pair_dot(inv_pa.reshape(NW, NCHUNK, 128),
                   inv_pb.reshape(NW, NCHUNK, 128), rows_a, rows_b)
  out = pl.pallas_call(
      _tc_body,
      out_shape=jax.ShapeDtypeStruct((1, 1), jnp.float32),
      out_specs=pl.BlockSpec(memory_space=pltpu.SMEM),
  )(part)
  return out[0, 0]


# speedup vs baseline: 32.1062x; 1.7245x over previous
import functools

import jax
import jax.numpy as jnp
from jax import lax
from jax.experimental import pallas as pl
from jax.experimental.pallas import tpu as pltpu
from jax.experimental.pallas import tpu_sc as plsc

VOCAB = 1000000
EMBED = 64
BATCH = 16384

NC = 2
NS = 16
L = 16
NW = NC * NS
BPW = BATCH // NW
NGRP = BPW // L
NCHUNK = BPW // 128


NRUN = BPW + L


NBUF = 4


def _stripe_body(sa_hbm, sb_hbm, rta_hbm, rsa_hbm, na_hbm,
                 rtb_hbm, rsb_hbm, nb_hbm, at_hbm, bt_hbm,
                 rowsa_hbm, rowsb_hbm,
                 sidx, rtc_v, rst_v, n_v, s0, s1, s2, s3, rows_v,
                 sem0, sem1, sem2, sem3):
  slots = (s0, s1, s2, s3)
  sems = (sem0, sem1, sem2, sem3)
  wid = lax.axis_index("s") * NC + lax.axis_index("c")
  lanes = lax.iota(jnp.int32, L)

  def vscal(chunk, lane):
    return jnp.max(jnp.where(lanes == lane, chunk, 0))

  def one_table(idx_hbm, rt_hbm, rs_hbm, n_hbm, xt_hbm, rows_hbm):
    pltpu.sync_copy(idx_hbm.at[wid], sidx)
    pltpu.sync_copy(rt_hbm.at[wid], rtc_v)
    pltpu.sync_copy(rs_hbm.at[wid], rst_v)
    pltpu.sync_copy(n_hbm.at[wid], n_v)
    nrun = vscal(n_v[...], 0)

    def rscal(ref, r):
      base = pl.multiple_of(jnp.bitwise_and(r, -L), 8)
      return vscal(ref[pl.ds(base, L)], jnp.bitwise_and(r, L - 1))

    def fetch(r, slot_ref, sem):
      tc = rscal(rtc_v, r)
      off = pl.multiple_of(tc * 128, 128)
      pltpu.async_copy(xt_hbm.at[:, pl.ds(off, 128)], slot_ref, sem)

    def drain(slot_ref, sem):
      pltpu.make_async_copy(xt_hbm.at[:, pl.ds(0, 128)], slot_ref, sem).wait()

    def extract(r, slot_ref):
      p0 = rscal(rst_v, r)
      p1 = rscal(rst_v, r + 1)

      def pos_body(p, _):
        ivec = sidx[lax.shift_right_logical(p, 4), :]
        col = vscal(jnp.bitwise_and(ivec, 127), jnp.bitwise_and(p, L - 1))
        csplat = jnp.broadcast_to(col, (L,))
        for q in range(EMBED // L):
          v = plsc.load_gather(slot_ref, [lanes + q * L, csplat])
          rows_v[p, pl.ds(q * L, L)] = v
        return _

      lax.fori_loop(p0, p1, pos_body, None)

    for u in range(NBUF - 1):
      @pl.when(u < nrun)
      def _pf():
        fetch(u, slots[u], sems[u])

    def run_quad(t, _):
      for u in range(NBUF):
        r = NBUF * t + u
        pf = r + NBUF - 1

        @pl.when(pf < nrun)
        def _pf():
          fetch(pf, slots[(u + NBUF - 1) % NBUF], sems[(u + NBUF - 1) % NBUF])

        @pl.when(r < nrun)
        def _do():
          drain(slots[u], sems[u])
          extract(r, slots[u])

      return _

    lax.fori_loop(0, lax.div(nrun + (NBUF - 1), NBUF), run_quad, None)
    pltpu.sync_copy(rows_v, rows_hbm.at[pl.ds(wid * BPW, BPW)])

  one_table(sa_hbm, rta_hbm, rsa_hbm, na_hbm, at_hbm, rowsa_hbm)
  one_table(sb_hbm, rtb_hbm, rsb_hbm, nb_hbm, bt_hbm, rowsb_hbm)


_stripe_gather = pl.kernel(
    _stripe_body,
    out_type=(jax.ShapeDtypeStruct((BATCH, EMBED), jnp.float32),
              jax.ShapeDtypeStruct((BATCH, EMBED), jnp.float32)),
    mesh=plsc.VectorSubcoreMesh(core_axis_name="c", subcore_axis_name="s",
                                num_cores=NC, num_subcores=NS),
    scratch_types=[
        pltpu.VMEM((NGRP, L), jnp.int32),
        pltpu.VMEM((NRUN,), jnp.int32),
        pltpu.VMEM((NRUN,), jnp.int32),
        pltpu.VMEM((L,), jnp.int32),
        pltpu.VMEM((EMBED, 128), jnp.float32),
        pltpu.VMEM((EMBED, 128), jnp.float32),
        pltpu.VMEM((EMBED, 128), jnp.float32),
        pltpu.VMEM((EMBED, 128), jnp.float32),
        pltpu.VMEM((BPW, EMBED), jnp.float32),
        pltpu.SemaphoreType.DMA,
        pltpu.SemaphoreType.DMA,
        pltpu.SemaphoreType.DMA,
        pltpu.SemaphoreType.DMA,
    ],
    compiler_params=pltpu.CompilerParams(use_tc_tiling_on_sc=True,
                                         needs_layout_passes=False),
)


def _run_tables(sorted_idx):
  t2 = lax.shift_right_logical(sorted_idx, 7).reshape(NW, BPW)
  new = jnp.concatenate(
      [jnp.ones((NW, 1), bool), t2[:, 1:] != t2[:, :-1]], axis=1)
  nrun = jnp.sum(new, axis=1).astype(jnp.int32)
  p_ix = jnp.broadcast_to(jnp.arange(BPW, dtype=jnp.int32)[None], (NW, BPW))
  starts = jnp.sort(jnp.where(new, p_ix, BPW), axis=1)
  run_tc = jnp.take_along_axis(t2, jnp.minimum(starts, BPW - 1), axis=1)
  pad = jnp.full((NW, NRUN - BPW), BPW, jnp.int32)
  run_st = jnp.concatenate([starts, pad], axis=1)
  run_tc = jnp.concatenate([run_tc, jnp.zeros_like(pad)], axis=1)
  nrep = jnp.broadcast_to(nrun[:, None], (NW, L)).astype(jnp.int32)
  return run_tc, run_st, nrep


def _pair_body(ia_hbm, ib_hbm, rowsa_hbm, rowsb_hbm, part_hbm,
               ia_v, ib_v, arows, brows, part_v, sem):
  wid = lax.axis_index("s") * NC + lax.axis_index("c")
  pltpu.sync_copy(ia_hbm.at[wid], ia_v)
  pltpu.sync_copy(ib_hbm.at[wid], ib_v)
  copies = []
  for c in range(NCHUNK):
    sl = pl.ds(c * 128, 128)
    copies.append(pltpu.async_copy(rowsa_hbm.at[ia_v.at[c]], arows.at[sl], sem))
    copies.append(pltpu.async_copy(rowsb_hbm.at[ib_v.at[c]], brows.at[sl], sem))
  for cp in copies:
    cp.wait()

  def row_body(r, _):
    acc = arows[r, pl.ds(0, L)] * brows[r, pl.ds(0, L)]
    for q in range(1, EMBED // L):
      acc = acc + arows[r, pl.ds(q * L, L)] * brows[r, pl.ds(q * L, L)]
    part_v[r, :] = acc
    return _

  lax.fori_loop(0, BPW, row_body, None)
  pltpu.sync_copy(part_v, part_hbm.at[pl.ds(wid * BPW, BPW)])


_pair_dot = pl.kernel(
    _pair_body,
    out_type=jax.ShapeDtypeStruct((BATCH, L), jnp.float32),
    mesh=plsc.VectorSubcoreMesh(core_axis_name="c", subcore_axis_name="s",
                                num_cores=NC, num_subcores=NS),
    scratch_types=[
        pltpu.VMEM((NCHUNK, 128), jnp.int32),
        pltpu.VMEM((NCHUNK, 128), jnp.int32),
        pltpu.VMEM((BPW, EMBED), jnp.float32),
        pltpu.VMEM((BPW, EMBED), jnp.float32),
        pltpu.VMEM((BPW, L), jnp.float32),
        pltpu.SemaphoreType.DMA,
    ],
    compiler_params=pltpu.CompilerParams(use_tc_tiling_on_sc=False),
)


def _tc_body(part_ref, o_ref):
  s = jnp.sum(part_ref[...], axis=1)
  lp = jnp.minimum(s, 0.0) - jnp.log1p(jnp.exp(-jnp.abs(s)))
  o_ref[0, 0] = -jnp.sum(lp) / BATCH


def kernel(center, context, input_embed, output_embed):
  center = center.astype(jnp.int32)
  context = context.astype(jnp.int32)
  pa = jnp.argsort(center)
  pb = jnp.argsort(context)
  sa = jnp.take(center, pa)
  sb = jnp.take(context, pb)
  inv_pa = jnp.argsort(pa).astype(jnp.int32)
  inv_pb = jnp.argsort(pb).astype(jnp.int32)
  rta, rsa, na = _run_tables(sa)
  rtb, rsb, nb = _run_tables(sb)
  rows_a, rows_b = _stripe_gather(
      sa.reshape(NW, NGRP, L), sb.reshape(NW, NGRP, L),
      rta, rsa, na, rtb, rsb, nb,
      input_embed.T, output_embed.T)
  part = _pair_dot(inv_pa.reshape(NW, NCHUNK, 128),
                   inv_pb.reshape(NW, NCHUNK, 128), rows_a, rows_b)
  out = pl.pallas_call(
      _tc_body,
      out_shape=jax.ShapeDtypeStruct((1, 1), jnp.float32),
      out_specs=pl.BlockSpec(memory_space=pltpu.SMEM),
  )(part)
  return out[0, 0]
